# baseline (speedup 1.0000x reference)
import jax
import jax.numpy as jnp
from jax.experimental import pallas as pl
from jax.experimental.pallas import tpu as pltpu

_N = 1024
_H = _N // 2
_Q = _N // 4


def _matmul(xh, yh):
    return jax.lax.dot_general(
        xh, yh,
        dimension_numbers=(((1,), (1,)), ((), ())),
        preferred_element_type=jnp.float32,
    )


def _simt(xh, xnh, yh):
    num = _matmul(xh, yh)
    yn = jnp.sqrt(jnp.sum(yh * yh, axis=1))
    denom = jnp.maximum(xnh * yn[None, :], 1e-8)
    return num / denom, num, yn


def _diag_count(simt, num, xnr, yn, col_off):
    rows, cols = simt.shape
    row = jax.lax.broadcasted_iota(jnp.int32, (rows, cols), 0)
    col = jax.lax.broadcasted_iota(jnp.int32, (rows, cols), 1) + col_off
    dn = jnp.sum(jnp.where(row == col, num, 0.0), axis=0, keepdims=True)
    d = dn / jnp.maximum(xnr * yn[None, :], 1e-8)
    beats = (simt > d) | ((simt == d) & (row < col))
    return jnp.sum(jnp.where(beats, 1.0, 0.0), axis=0, keepdims=True), d


def _cnt(pred):
    return jnp.sum(jnp.where(pred, 1.0, 0.0), axis=0, keepdims=True)


def _accs(cnt):
    t1 = jnp.sum(jnp.where(cnt == 0.0, 1.0, 0.0), axis=1, keepdims=True)
    t10 = jnp.sum(jnp.where(cnt < 10.0, 1.0, 0.0), axis=1, keepdims=True)
    return t1, t10


def _acc_kernel(z_hbm, y_hbm, out_ref, xv, yv, s0, s1, s2, s3, s4):
    lo = pl.ds(0, _H)
    hi = pl.ds(_H, _H)
    qa = pl.ds(0, _Q)
    qb = pl.ds(_Q, _Q)
    cx0 = pltpu.make_async_copy(z_hbm.at[lo, :], xv.at[lo, :], s0)
    cx0.start()
    cya = pltpu.make_async_copy(y_hbm.at[qa, :], yv.at[qa, :], s1)
    cya.start()
    cyb = pltpu.make_async_copy(y_hbm.at[qb, :], yv.at[qb, :], s2)
    cyb.start()
    cx1 = pltpu.make_async_copy(z_hbm.at[hi, :], xv.at[hi, :], s3)
    cx1.start()
    cy1 = pltpu.make_async_copy(y_hbm.at[hi, :], yv.at[hi, :], s4)
    cy1.start()

    cx0.wait()
    x0 = xv[lo, :]
    xn0v = jnp.sqrt(jnp.sum(x0 * x0, axis=1))
    xn0 = xn0v[:, None]

    cya.wait()
    ya = yv[qa, :]
    sa, na, yna = _simt(x0, xn0, ya)

    cyb.wait()
    yb = yv[qb, :]
    sb, nb, ynb = _simt(x0, xn0, yb)

    cx1.wait()
    x1 = xv[hi, :]
    xn1v = jnp.sqrt(jnp.sum(x1 * x1, axis=1))
    xn1 = xn1v[:, None]
    s10a, _, _ = _simt(x1, xn1, ya)
    s10b, _, _ = _simt(x1, xn1, yb)

    cy1.wait()
    y1 = yv[hi, :]
    s11, n11, yn1 = _simt(x1, xn1, y1)
    s01, _, _ = _simt(x0, xn0, y1)

    ca, da = _diag_count(sa, na, xn0v[None, 0:_Q], yna, 0)
    cnta = ca + _cnt(s10a > da)
    t1, t10 = _accs(cnta)

    cb, db = _diag_count(sb, nb, xn0v[None, _Q:_H], ynb, _Q)
    cntb = cb + _cnt(s10b > db)
    t1b, t10b = _accs(cntb)
    t1 = t1 + t1b
    t10 = t10 + t10b

    c11, d1 = _diag_count(s11, n11, xn1v[None, :], yn1, 0)
    cnt1 = c11 + _cnt(s01 >= d1)
    t1c, t10c = _accs(cnt1)
    t1 = t1 + t1c
    t10 = t10 + t10c

    out_ref[...] = jnp.concatenate([t1, t10], axis=1) * (1.0 / _N)


def kernel(Z, Y):
    out = pl.pallas_call(
        _acc_kernel,
        in_specs=[
            pl.BlockSpec(memory_space=pltpu.MemorySpace.HBM),
            pl.BlockSpec(memory_space=pltpu.MemorySpace.HBM),
        ],
        out_specs=pl.BlockSpec(memory_space=pltpu.MemorySpace.VMEM),
        out_shape=jax.ShapeDtypeStruct((1, 2), jnp.float32),
        scratch_shapes=[
            pltpu.VMEM((_N, _N), jnp.float32),
            pltpu.VMEM((_N, _N), jnp.float32),
        ] + [pltpu.SemaphoreType.DMA] * 5,
    )(Z, Y)
    return (out[0, 0], out[0, 1])


# speedup vs baseline: 1.0593x; 1.0593x over previous
import jax
import jax.numpy as jnp
from jax.experimental import pallas as pl
from jax.experimental.pallas import tpu as pltpu

_N = 1024
_H = _N // 2


def _simt(xh, xnh, yh):
    num = jax.lax.dot_general(
        xh, yh,
        dimension_numbers=(((1,), (1,)), ((), ())),
        preferred_element_type=jnp.float32,
    )
    yn = jnp.sqrt(jnp.sum(yh * yh, axis=1))
    denom = jnp.maximum(xnh * yn[None, :], 1e-8)
    return num / denom


def _diag_count(simt):
    row = jax.lax.broadcasted_iota(jnp.int32, (_H, _H), 0)
    col = jax.lax.broadcasted_iota(jnp.int32, (_H, _H), 1)
    d = jnp.sum(jnp.where(row == col, simt, 0.0), axis=0, keepdims=True)
    beats = (simt > d) | ((simt == d) & (row < col))
    return jnp.sum(jnp.where(beats, 1.0, 0.0), axis=0, keepdims=True), d


def _cnt(pred):
    return jnp.sum(jnp.where(pred, 1.0, 0.0), axis=0, keepdims=True)


def _accs(cnt):
    t1 = jnp.sum(jnp.where(cnt == 0.0, 1.0, 0.0), axis=1, keepdims=True)
    t10 = jnp.sum(jnp.where(cnt < 10.0, 1.0, 0.0), axis=1, keepdims=True)
    return t1, t10


def _acc_kernel(z_hbm, y_hbm, out_ref, xv, yv, sx0, sx1, sy0, sy1):
    lo = pl.ds(0, _H)
    hi = pl.ds(_H, _H)
    cx0 = pltpu.make_async_copy(z_hbm.at[lo, :], xv.at[lo, :], sx0)
    cx0.start()
    cy0 = pltpu.make_async_copy(y_hbm.at[lo, :], yv.at[lo, :], sy0)
    cy0.start()
    cx1 = pltpu.make_async_copy(z_hbm.at[hi, :], xv.at[hi, :], sx1)
    cx1.start()
    cy1 = pltpu.make_async_copy(y_hbm.at[hi, :], yv.at[hi, :], sy1)
    cy1.start()

    cx0.wait()
    x0 = xv[lo, :]
    xn0 = jnp.sqrt(jnp.sum(x0 * x0, axis=1))[:, None]

    cy0.wait()
    y0 = yv[lo, :]
    s00 = _simt(x0, xn0, y0)

    cx1.wait()
    x1 = xv[hi, :]
    xn1 = jnp.sqrt(jnp.sum(x1 * x1, axis=1))[:, None]
    s10 = _simt(x1, xn1, y0)

    cy1.wait()
    y1 = yv[hi, :]
    s11 = _simt(x1, xn1, y1)
    s01 = _simt(x0, xn0, y1)

    c00, d0 = _diag_count(s00)
    cnt0 = c00 + _cnt(s10 > d0)
    t1a, t10a = _accs(cnt0)

    c11, d1 = _diag_count(s11)
    cnt1 = c11 + _cnt(s01 >= d1)
    t1b, t10b = _accs(cnt1)

    out_ref[...] = jnp.concatenate(
        [t1a + t1b, t10a + t10b], axis=1
    ) * (1.0 / _N)


def kernel(Z, Y):
    out = pl.pallas_call(
        _acc_kernel,
        in_specs=[
            pl.BlockSpec(memory_space=pltpu.MemorySpace.HBM),
            pl.BlockSpec(memory_space=pltpu.MemorySpace.HBM),
        ],
        out_specs=pl.BlockSpec(memory_space=pltpu.MemorySpace.VMEM),
        out_shape=jax.ShapeDtypeStruct((1, 2), jnp.float32),
        scratch_shapes=[
            pltpu.VMEM((_N, _N), jnp.float32),
            pltpu.VMEM((_N, _N), jnp.float32),
            pltpu.SemaphoreType.DMA,
            pltpu.SemaphoreType.DMA,
            pltpu.SemaphoreType.DMA,
            pltpu.SemaphoreType.DMA,
        ],
    )(Z, Y)
    return (out[0, 0], out[0, 1])
